# Initial kernel scaffold; baseline (speedup 1.0000x reference)
import jax
import jax.numpy as jnp
from jax.experimental import pallas as pl


def kernel(atom_fea, nbr_fea, self_idx, nbr_idx, W_emb, b_emb, fc_W, fc_b, bn1_g, bn1_b, bn2_g, bn2_b):
    raise NotImplementedError("write your pallas kernel here")


# speedup vs baseline: 1.9135x; 1.9135x over previous
import functools

import jax
import jax.numpy as jnp
from jax import lax
from jax.experimental import pallas as pl
from jax.experimental.pallas import tpu as pltpu
from jax.experimental.pallas import tpu_sc as plsc

F32 = jnp.float32
NW = 32
EPS = 1e-5


def _sc_gather(P1, P2, ia2d, ib2d, E):
    n_units = E // 128
    rounds = (n_units + NW - 1) // NW
    C = P1.shape[1]
    mesh = plsc.VectorSubcoreMesh(core_axis_name="c", subcore_axis_name="s")

    @functools.partial(
        pl.kernel,
        out_type=(
            jax.ShapeDtypeStruct((E, C), F32),
            jax.ShapeDtypeStruct((E, C), F32),
        ),
        mesh=mesh,
        scratch_types=[
            pltpu.VMEM((128,), jnp.int32),
            pltpu.VMEM((128,), jnp.int32),
            pltpu.VMEM((128, C), F32),
            pltpu.VMEM((128, C), F32),
            pltpu.SemaphoreType.DMA,
            pltpu.SemaphoreType.DMA,
        ],
    )
    def k(p1_hbm, p2_hbm, ia_hbm, ib_hbm, o1_hbm, o2_hbm,
          ia_v, ib_v, g1_v, g2_v, sem1, sem2):
        wid = lax.axis_index("s") * 2 + lax.axis_index("c")

        def body(r, _):
            uid = r * NW + wid

            @pl.when(uid < n_units)
            def _():
                pltpu.sync_copy(ia_hbm.at[uid], ia_v)
                pltpu.sync_copy(ib_hbm.at[uid], ib_v)
                cp1 = pltpu.async_copy(p1_hbm.at[ia_v], g1_v, sem1)
                cp2 = pltpu.async_copy(p2_hbm.at[ib_v], g2_v, sem2)
                cp1.wait()
                cp2.wait()
                pltpu.sync_copy(g1_v, o1_hbm.at[pl.ds(uid * 128, 128)])
                pltpu.sync_copy(g2_v, o2_hbm.at[pl.ds(uid * 128, 128)])
            return 0

        lax.fori_loop(0, rounds, body, 0)

    return k(P1, P2, ia2d, ib2d)


def _sc_scatter(msg, ia2d, zeros_u, N, E):
    n_units = E // 128
    rounds = (n_units + NW - 1) // NW
    C = msg.shape[1]
    mesh = plsc.VectorSubcoreMesh(core_axis_name="c", subcore_axis_name="s")

    @functools.partial(
        pl.kernel,
        out_type=jax.ShapeDtypeStruct((2, N, 128), F32),
        mesh=mesh,
        scratch_types=[
            pltpu.VMEM((128,), jnp.int32),
            pltpu.VMEM((128, 128), F32),
            pltpu.VMEM_SHARED((N, 128), F32),
        ],
    )
    def k(msg_hbm, ia_hbm, z_hbm, out_hbm, ia_v, m_v, u_sh):
        cid = lax.axis_index("c")
        sid = lax.axis_index("s")
        wid = sid * 2 + cid

        @pl.when(sid == 0)
        def _():
            pltpu.sync_copy(z_hbm, u_sh)

        plsc.subcore_barrier()

        def body(r, _):
            uid = r * NW + wid

            @pl.when(uid < n_units)
            def _():
                pltpu.sync_copy(ia_hbm.at[uid], ia_v)
                pltpu.sync_copy(msg_hbm.at[pl.ds(uid * 128, 128)], m_v)
                pltpu.sync_copy(m_v, u_sh.at[ia_v], add=True)
            return 0

        lax.fori_loop(0, rounds, body, 0)
        plsc.subcore_barrier()

        @pl.when(sid == 0)
        def _():
            pltpu.sync_copy(u_sh, out_hbm.at[cid])

    return k(msg, ia2d, zeros_u)


def _embed(atom_fea, W_emb, b_emb, W1, W2):
    N, EMB = atom_fea.shape
    F = W_emb.shape[1]
    TF = W1.shape[1]
    BR = 2000

    def body(a_ref, we_ref, be_ref, w1_ref, w2_ref, x_ref, p1_ref, p2_ref):
        x = jnp.dot(a_ref[...], we_ref[...], preferred_element_type=F32)
        x = x + be_ref[0:1, :]
        x_ref[...] = x
        p1_ref[...] = jnp.dot(x, w1_ref[...], preferred_element_type=F32)
        p2_ref[...] = jnp.dot(x, w2_ref[...], preferred_element_type=F32)

    return pl.pallas_call(
        body,
        grid=(N // BR,),
        in_specs=[
            pl.BlockSpec((BR, EMB), lambda i: (i, 0)),
            pl.BlockSpec((EMB, F), lambda i: (0, 0)),
            pl.BlockSpec((8, F), lambda i: (0, 0)),
            pl.BlockSpec((F, TF), lambda i: (0, 0)),
            pl.BlockSpec((F, TF), lambda i: (0, 0)),
        ],
        out_specs=[
            pl.BlockSpec((BR, F), lambda i: (i, 0)),
            pl.BlockSpec((BR, TF), lambda i: (i, 0)),
            pl.BlockSpec((BR, TF), lambda i: (i, 0)),
        ],
        out_shape=[
            jax.ShapeDtypeStruct((N, F), F32),
            jax.ShapeDtypeStruct((N, TF), F32),
            jax.ShapeDtypeStruct((N, TF), F32),
        ],
    )(atom_fea, W_emb, b_emb, W1, W2)


def _stats(G1, G2, nbr_fea, W3, b, E):
    TF = G1.shape[1]
    NB = nbr_fea.shape[1]
    BE = 2000

    def body(g1_ref, g2_ref, nf_ref, w3_ref, b_ref, s1_ref, s2_ref):
        t = (g1_ref[...] + g2_ref[...]
             + jnp.dot(nf_ref[...], w3_ref[...], preferred_element_type=F32)
             + b_ref[0:1, :])
        p1 = jnp.sum(t.reshape(BE // 8, 8, TF), axis=0)
        p2 = jnp.sum((t * t).reshape(BE // 8, 8, TF), axis=0)

        @pl.when(pl.program_id(0) == 0)
        def _():
            s1_ref[...] = jnp.zeros_like(s1_ref)
            s2_ref[...] = jnp.zeros_like(s2_ref)

        s1_ref[...] += p1
        s2_ref[...] += p2

    return pl.pallas_call(
        body,
        grid=(E // BE,),
        in_specs=[
            pl.BlockSpec((BE, TF), lambda i: (i, 0)),
            pl.BlockSpec((BE, TF), lambda i: (i, 0)),
            pl.BlockSpec((BE, NB), lambda i: (i, 0)),
            pl.BlockSpec((NB, TF), lambda i: (0, 0)),
            pl.BlockSpec((8, TF), lambda i: (0, 0)),
        ],
        out_specs=[
            pl.BlockSpec((8, TF), lambda i: (0, 0)),
            pl.BlockSpec((8, TF), lambda i: (0, 0)),
        ],
        out_shape=[
            jax.ShapeDtypeStruct((8, TF), F32),
            jax.ShapeDtypeStruct((8, TF), F32),
        ],
    )(G1, G2, nbr_fea, W3, b)


def _gate(G1, G2, nbr_fea, W3, b, scale, shift, E):
    TF = G1.shape[1]
    NB = nbr_fea.shape[1]
    F = TF // 2
    BE = 2000

    def body(g1_ref, g2_ref, nf_ref, w3_ref, b_ref, sc_ref, sh_ref, m_ref):
        t = (g1_ref[...] + g2_ref[...]
             + jnp.dot(nf_ref[...], w3_ref[...], preferred_element_type=F32)
             + b_ref[0:1, :])
        t = t * sc_ref[0:1, :] + sh_ref[0:1, :]
        LOG2E = 1.4426950408889634
        LN2 = 0.6931471805599453
        f = t[:, :F]
        c = t[:, F:]
        filt = 1.0 / (1.0 + jnp.exp2(-LOG2E * f))
        core = jnp.maximum(c, 0.0) + LN2 * jnp.log2(
            1.0 + jnp.exp2(-LOG2E * jnp.abs(c)))
        m_ref[...] = jnp.concatenate(
            [filt * core, jnp.zeros_like(filt)], axis=1)

    return pl.pallas_call(
        body,
        grid=(E // BE,),
        in_specs=[
            pl.BlockSpec((BE, TF), lambda i: (i, 0)),
            pl.BlockSpec((BE, TF), lambda i: (i, 0)),
            pl.BlockSpec((BE, NB), lambda i: (i, 0)),
            pl.BlockSpec((NB, TF), lambda i: (0, 0)),
            pl.BlockSpec((8, TF), lambda i: (0, 0)),
            pl.BlockSpec((8, TF), lambda i: (0, 0)),
            pl.BlockSpec((8, TF), lambda i: (0, 0)),
        ],
        out_specs=pl.BlockSpec((BE, TF), lambda i: (i, 0)),
        out_shape=jax.ShapeDtypeStruct((E, TF), F32),
    )(G1, G2, nbr_fea, W3, b, scale, shift)


def _finish(Uparts, x, g2, b2, W1, W2, last):
    N, F = x.shape
    TF = W1.shape[1]

    def body(u_ref, x_ref, g_ref, b_ref, w1_ref, w2_ref, *outs):
        u = u_ref[0, :, :F] + u_ref[1, :, :F]
        mean = jnp.mean(u, axis=0, keepdims=True)
        var = jnp.mean(u * u, axis=0, keepdims=True) - mean * mean
        un = g_ref[0:1, :] * (u - mean) * jax.lax.rsqrt(var + EPS) + b_ref[0:1, :]
        xn = jax.nn.softplus(x_ref[...] + un)
        outs[0][...] = xn
        if not last:
            outs[1][...] = jnp.dot(xn, w1_ref[...], preferred_element_type=F32)
            outs[2][...] = jnp.dot(xn, w2_ref[...], preferred_element_type=F32)

    out_shape = [jax.ShapeDtypeStruct((N, F), F32)]
    if not last:
        out_shape += [jax.ShapeDtypeStruct((N, TF), F32),
                      jax.ShapeDtypeStruct((N, TF), F32)]
    return pl.pallas_call(body, out_shape=out_shape)(Uparts, x, g2, b2, W1, W2)


def kernel(atom_fea, nbr_fea, self_idx, nbr_idx, W_emb, b_emb, fc_W, fc_b,
           bn1_g, bn1_b, bn2_g, bn2_b):
    N = atom_fea.shape[0]
    E = nbr_fea.shape[0]
    F = W_emb.shape[1]
    TF = 2 * F
    n_layers = fc_W.shape[0]

    W1 = fc_W[:, :F, :]
    W2 = fc_W[:, F:TF, :]
    W3 = fc_W[:, TF:, :]

    def row8(v):
        return jnp.broadcast_to(v.reshape(1, -1), (8, v.shape[0])).astype(F32)

    ia2d = self_idx.reshape(E // 128, 128)
    ib2d = nbr_idx.reshape(E // 128, 128)
    zeros_u = jnp.zeros((N, 128), F32)

    x, P1, P2 = _embed(atom_fea, W_emb, row8(b_emb), W1[0], W2[0])

    for i in range(n_layers):
        G1, G2 = _sc_gather(P1, P2, ia2d, ib2d, E)
        s1, s2 = _stats(G1, G2, nbr_fea, W3[i], row8(fc_b[i]), E)
        mean = jnp.sum(s1, axis=0) / E
        var = jnp.sum(s2, axis=0) / E - mean * mean
        scale = bn1_g[i] * jax.lax.rsqrt(var + EPS)
        shift = bn1_b[i] - mean * scale
        msg = _gate(G1, G2, nbr_fea, W3[i], row8(fc_b[i]),
                    row8(scale), row8(shift), E)
        Uparts = _sc_scatter(msg, ia2d, zeros_u, N, E)
        last = i == n_layers - 1
        nxt = 0 if last else i + 1
        outs = _finish(Uparts, x, row8(bn2_g[i]), row8(bn2_b[i]),
                       W1[nxt], W2[nxt], last)
        if last:
            x = outs[0]
        else:
            x, P1, P2 = outs
    return x


# speedup vs baseline: 2.4469x; 1.2788x over previous
import functools

import jax
import jax.numpy as jnp
from jax import lax
from jax.experimental import pallas as pl
from jax.experimental.pallas import tpu as pltpu
from jax.experimental.pallas import tpu_sc as plsc

F32 = jnp.float32
NW = 32
EPS = 1e-5


def _sc_gather(P1, P2, ia2d, ib2d, E):
    n_units = E // 128
    UPT = ((n_units + NW - 1) // NW + 7) // 8 * 8
    C = P1.shape[1]
    mesh = plsc.VectorSubcoreMesh(core_axis_name="c", subcore_axis_name="s")

    @functools.partial(
        pl.kernel,
        out_type=jax.ShapeDtypeStruct((E, C), F32),
        mesh=mesh,
        scratch_types=[
            pltpu.VMEM((UPT, 128), jnp.int32),
            pltpu.VMEM((UPT, 128), jnp.int32),
            pltpu.VMEM((128, C), F32),
            pltpu.VMEM((128, C), F32),
            pltpu.VMEM((128, C), F32),
            pltpu.VMEM((128, C), F32),
            pltpu.SemaphoreType.DMA,
            pltpu.SemaphoreType.DMA,
            pltpu.SemaphoreType.DMA,
            pltpu.SemaphoreType.DMA,
        ],
    )
    def k(p1_hbm, p2_hbm, ia_hbm, ib_hbm, o_hbm,
          ia_v, ib_v, g1a, g2a, g1b, g2b, sga, sgb, swa, swb):
        wid = lax.axis_index("s") * 2 + lax.axis_index("c")
        start = wid * UPT
        cnt = jnp.minimum(jnp.maximum(n_units - start, 0), UPT)

        @pl.when(cnt > 0)
        def _():
            pltpu.sync_copy(ia_hbm.at[pl.ds(start, UPT)], ia_v)
            pltpu.sync_copy(ib_hbm.at[pl.ds(start, UPT)], ib_v)

        def addto(dst, src):
            def arow(r, _):
                for kk in range(C // 16):
                    sl = pl.ds(16 * kk, 16)
                    dst[r, sl] = dst[r, sl] + src[r, sl]
                return 0
            lax.fori_loop(0, 128, arow, 0)

        def phase(j, g1, g2, sg, sw, first):
            uid = start + j

            @pl.when(j < cnt)
            def _():
                @pl.when(jnp.logical_not(first))
                def _():
                    pltpu.make_async_copy(
                        g1, o_hbm.at[pl.ds(0, 128)], sw).wait()
                pltpu.async_copy(p1_hbm.at[ia_v.at[j]], g1, sg)
                pltpu.async_copy(p2_hbm.at[ib_v.at[j]], g2, sg)

        def finish_phase(j, g1, g2, sg, sw):
            uid = start + j

            @pl.when(j < cnt)
            def _():
                pltpu.make_async_copy(p1_hbm.at[ia_v.at[j]], g1, sg).wait()
                pltpu.make_async_copy(p2_hbm.at[ib_v.at[j]], g2, sg).wait()
                addto(g1, g2)
                pltpu.async_copy(g1, o_hbm.at[pl.ds(uid * 128, 128)], sw)

        def body(r, _):
            j0 = 2 * r
            j1 = 2 * r + 1
            phase(j0, g1a, g2a, sga, swa, r == 0)
            phase(j1, g1b, g2b, sgb, swb, r == 0)
            finish_phase(j0, g1a, g2a, sga, swa)
            finish_phase(j1, g1b, g2b, sgb, swb)
            return 0

        lax.fori_loop(0, UPT // 2, body, 0)

        @pl.when(cnt > 0)
        def _():
            pltpu.make_async_copy(g1a, o_hbm.at[pl.ds(0, 128)], swa).wait()

        @pl.when(cnt > 1)
        def _():
            pltpu.make_async_copy(g1b, o_hbm.at[pl.ds(0, 128)], swb).wait()

    return k(P1, P2, ia2d, ib2d)


def _sc_scatter(msg, ia2d, zeros_u, N, E):
    n_units = E // 128
    rounds = (n_units + NW - 1) // NW
    C = msg.shape[1]
    mesh = plsc.VectorSubcoreMesh(core_axis_name="c", subcore_axis_name="s")

    @functools.partial(
        pl.kernel,
        out_type=jax.ShapeDtypeStruct((2, N, 128), F32),
        mesh=mesh,
        scratch_types=[
            pltpu.VMEM((128,), jnp.int32),
            pltpu.VMEM((128, 128), F32),
            pltpu.VMEM_SHARED((N, 128), F32),
        ],
    )
    def k(msg_hbm, ia_hbm, z_hbm, out_hbm, ia_v, m_v, u_sh):
        cid = lax.axis_index("c")
        sid = lax.axis_index("s")
        wid = sid * 2 + cid

        @pl.when(sid == 0)
        def _():
            pltpu.sync_copy(z_hbm, u_sh)

        plsc.subcore_barrier()

        def body(r, _):
            uid = r * NW + wid

            @pl.when(uid < n_units)
            def _():
                pltpu.sync_copy(ia_hbm.at[uid], ia_v)
                pltpu.sync_copy(msg_hbm.at[pl.ds(uid * 128, 128)], m_v)
                pltpu.sync_copy(m_v, u_sh.at[ia_v], add=True)
            return 0

        lax.fori_loop(0, rounds, body, 0)
        plsc.subcore_barrier()

        @pl.when(sid == 0)
        def _():
            pltpu.sync_copy(u_sh, out_hbm.at[cid])

    return k(msg, ia2d, zeros_u)


def _embed(atom_fea, W_emb, b_emb, W1, W2):
    N, EMB = atom_fea.shape
    F = W_emb.shape[1]
    TF = W1.shape[1]
    BR = 2000

    def body(a_ref, we_ref, be_ref, w1_ref, w2_ref, x_ref, p1_ref, p2_ref):
        x = jnp.dot(a_ref[...], we_ref[...], preferred_element_type=F32)
        x = x + be_ref[0:1, :]
        x_ref[...] = x
        p1_ref[...] = jnp.dot(x, w1_ref[...], preferred_element_type=F32)
        p2_ref[...] = jnp.dot(x, w2_ref[...], preferred_element_type=F32)

    return pl.pallas_call(
        body,
        grid=(N // BR,),
        in_specs=[
            pl.BlockSpec((BR, EMB), lambda i: (i, 0)),
            pl.BlockSpec((EMB, F), lambda i: (0, 0)),
            pl.BlockSpec((8, F), lambda i: (0, 0)),
            pl.BlockSpec((F, TF), lambda i: (0, 0)),
            pl.BlockSpec((F, TF), lambda i: (0, 0)),
        ],
        out_specs=[
            pl.BlockSpec((BR, F), lambda i: (i, 0)),
            pl.BlockSpec((BR, TF), lambda i: (i, 0)),
            pl.BlockSpec((BR, TF), lambda i: (i, 0)),
        ],
        out_shape=[
            jax.ShapeDtypeStruct((N, F), F32),
            jax.ShapeDtypeStruct((N, TF), F32),
            jax.ShapeDtypeStruct((N, TF), F32),
        ],
    )(atom_fea, W_emb, b_emb, W1, W2)


def _stats(G12, nbr_fea, W3, b, E):
    TF = G12.shape[1]
    NB = nbr_fea.shape[1]
    BE = 2000

    def body(g_ref, nf_ref, w3_ref, b_ref, s1_ref, s2_ref):
        t = (g_ref[...]
             + jnp.dot(nf_ref[...], w3_ref[...], preferred_element_type=F32)
             + b_ref[0:1, :])
        p1 = jnp.sum(t.reshape(BE // 8, 8, TF), axis=0)
        p2 = jnp.sum((t * t).reshape(BE // 8, 8, TF), axis=0)

        @pl.when(pl.program_id(0) == 0)
        def _():
            s1_ref[...] = jnp.zeros_like(s1_ref)
            s2_ref[...] = jnp.zeros_like(s2_ref)

        s1_ref[...] += p1
        s2_ref[...] += p2

    return pl.pallas_call(
        body,
        grid=(E // BE,),
        in_specs=[
            pl.BlockSpec((BE, TF), lambda i: (i, 0)),
            pl.BlockSpec((BE, NB), lambda i: (i, 0)),
            pl.BlockSpec((NB, TF), lambda i: (0, 0)),
            pl.BlockSpec((8, TF), lambda i: (0, 0)),
        ],
        out_specs=[
            pl.BlockSpec((8, TF), lambda i: (0, 0)),
            pl.BlockSpec((8, TF), lambda i: (0, 0)),
        ],
        out_shape=[
            jax.ShapeDtypeStruct((8, TF), F32),
            jax.ShapeDtypeStruct((8, TF), F32),
        ],
    )(G12, nbr_fea, W3, b)


def _gate(G12, nbr_fea, W3, b, scale, shift, E):
    TF = G12.shape[1]
    NB = nbr_fea.shape[1]
    F = TF // 2
    BE = 2000

    def body(g_ref, nf_ref, w3_ref, b_ref, sc_ref, sh_ref, m_ref):
        t = (g_ref[...]
             + jnp.dot(nf_ref[...], w3_ref[...], preferred_element_type=F32)
             + b_ref[0:1, :])
        t = t * sc_ref[0:1, :] + sh_ref[0:1, :]
        LOG2E = 1.4426950408889634
        LN2 = 0.6931471805599453
        f = t[:, :F]
        c = t[:, F:]
        filt = 1.0 / (1.0 + jnp.exp2(-LOG2E * f))
        core = jnp.maximum(c, 0.0) + LN2 * jnp.log2(
            1.0 + jnp.exp2(-LOG2E * jnp.abs(c)))
        m_ref[...] = jnp.concatenate(
            [filt * core, jnp.zeros_like(filt)], axis=1)

    return pl.pallas_call(
        body,
        grid=(E // BE,),
        in_specs=[
            pl.BlockSpec((BE, TF), lambda i: (i, 0)),
            pl.BlockSpec((BE, NB), lambda i: (i, 0)),
            pl.BlockSpec((NB, TF), lambda i: (0, 0)),
            pl.BlockSpec((8, TF), lambda i: (0, 0)),
            pl.BlockSpec((8, TF), lambda i: (0, 0)),
            pl.BlockSpec((8, TF), lambda i: (0, 0)),
        ],
        out_specs=pl.BlockSpec((BE, TF), lambda i: (i, 0)),
        out_shape=jax.ShapeDtypeStruct((E, TF), F32),
    )(G12, nbr_fea, W3, b, scale, shift)


def _finish(Uparts, x, g2, b2, W1, W2, last):
    N, F = x.shape
    TF = W1.shape[1]

    def body(u_ref, x_ref, g_ref, b_ref, w1_ref, w2_ref, *outs):
        u = u_ref[0, :, :F] + u_ref[1, :, :F]
        mean = jnp.mean(u, axis=0, keepdims=True)
        var = jnp.mean(u * u, axis=0, keepdims=True) - mean * mean
        un = g_ref[0:1, :] * (u - mean) * jax.lax.rsqrt(var + EPS) + b_ref[0:1, :]
        xn = jax.nn.softplus(x_ref[...] + un)
        outs[0][...] = xn
        if not last:
            outs[1][...] = jnp.dot(xn, w1_ref[...], preferred_element_type=F32)
            outs[2][...] = jnp.dot(xn, w2_ref[...], preferred_element_type=F32)

    out_shape = [jax.ShapeDtypeStruct((N, F), F32)]
    if not last:
        out_shape += [jax.ShapeDtypeStruct((N, TF), F32),
                      jax.ShapeDtypeStruct((N, TF), F32)]
    return pl.pallas_call(body, out_shape=out_shape)(Uparts, x, g2, b2, W1, W2)


def kernel(atom_fea, nbr_fea, self_idx, nbr_idx, W_emb, b_emb, fc_W, fc_b,
           bn1_g, bn1_b, bn2_g, bn2_b):
    N = atom_fea.shape[0]
    E = nbr_fea.shape[0]
    F = W_emb.shape[1]
    TF = 2 * F
    n_layers = fc_W.shape[0]

    W1 = fc_W[:, :F, :]
    W2 = fc_W[:, F:TF, :]
    W3 = fc_W[:, TF:, :]

    def row8(v):
        return jnp.broadcast_to(v.reshape(1, -1), (8, v.shape[0])).astype(F32)

    ia2d = self_idx.reshape(E // 128, 128)
    ib2d = nbr_idx.reshape(E // 128, 128)
    zeros_u = jnp.zeros((N, 128), F32)

    x, P1, P2 = _embed(atom_fea, W_emb, row8(b_emb), W1[0], W2[0])

    for i in range(n_layers):
        G12 = _sc_gather(P1, P2, ia2d, ib2d, E)
        s1, s2 = _stats(G12, nbr_fea, W3[i], row8(fc_b[i]), E)
        mean = jnp.sum(s1, axis=0) / E
        var = jnp.sum(s2, axis=0) / E - mean * mean
        scale = bn1_g[i] * jax.lax.rsqrt(var + EPS)
        shift = bn1_b[i] - mean * scale
        msg = _gate(G12, nbr_fea, W3[i], row8(fc_b[i]),
                    row8(scale), row8(shift), E)
        Uparts = _sc_scatter(msg, ia2d, zeros_u, N, E)
        last = i == n_layers - 1
        nxt = 0 if last else i + 1
        outs = _finish(Uparts, x, row8(bn2_g[i]), row8(bn2_b[i]),
                       W1[nxt], W2[nxt], last)
        if last:
            x = outs[0]
        else:
            x, P1, P2 = outs
    return x


# speedup vs baseline: 2.6470x; 1.0818x over previous
import functools

import jax
import jax.numpy as jnp
from jax import lax
from jax.experimental import pallas as pl
from jax.experimental.pallas import tpu as pltpu
from jax.experimental.pallas import tpu_sc as plsc

F32 = jnp.float32
NW = 32
EPS = 1e-5


def _sc_gather(P1, P2, ia2d, ib2d, E):
    n_units = E // 128
    UPT = ((n_units + NW - 1) // NW + 7) // 8 * 8
    C = P1.shape[1]
    mesh = plsc.VectorSubcoreMesh(core_axis_name="c", subcore_axis_name="s")

    @functools.partial(
        pl.kernel,
        out_type=jax.ShapeDtypeStruct((E, C), F32),
        mesh=mesh,
        scratch_types=[
            pltpu.VMEM((UPT, 128), jnp.int32),
            pltpu.VMEM((UPT, 128), jnp.int32),
            pltpu.VMEM((128, C), F32),
            pltpu.VMEM((128, C), F32),
            pltpu.VMEM((128, C), F32),
            pltpu.VMEM((128, C), F32),
            pltpu.SemaphoreType.DMA,
            pltpu.SemaphoreType.DMA,
            pltpu.SemaphoreType.DMA,
            pltpu.SemaphoreType.DMA,
        ],
    )
    def k(p1_hbm, p2_hbm, ia_hbm, ib_hbm, o_hbm,
          ia_v, ib_v, g1a, g2a, g1b, g2b, sga, sgb, swa, swb):
        wid = lax.axis_index("s") * 2 + lax.axis_index("c")
        start = wid * UPT
        cnt = jnp.minimum(jnp.maximum(n_units - start, 0), UPT)

        @pl.when(cnt > 0)
        def _():
            pltpu.sync_copy(ia_hbm.at[pl.ds(start, UPT)], ia_v)
            pltpu.sync_copy(ib_hbm.at[pl.ds(start, UPT)], ib_v)

        def addto(dst, src):
            def arow(r, _):
                for kk in range(C // 16):
                    sl = pl.ds(16 * kk, 16)
                    dst[r, sl] = dst[r, sl] + src[r, sl]
                return 0
            lax.fori_loop(0, 128, arow, 0)

        def phase(j, g1, g2, sg, sw, first):
            uid = start + j

            @pl.when(j < cnt)
            def _():
                @pl.when(jnp.logical_not(first))
                def _():
                    pltpu.make_async_copy(
                        g1, o_hbm.at[pl.ds(0, 128)], sw).wait()
                pltpu.async_copy(p1_hbm.at[ia_v.at[j]], g1, sg)
                pltpu.async_copy(p2_hbm.at[ib_v.at[j]], g2, sg)

        def finish_phase(j, g1, g2, sg, sw):
            uid = start + j

            @pl.when(j < cnt)
            def _():
                pltpu.make_async_copy(p1_hbm.at[ia_v.at[j]], g1, sg).wait()
                pltpu.make_async_copy(p2_hbm.at[ib_v.at[j]], g2, sg).wait()
                addto(g1, g2)
                pltpu.async_copy(g1, o_hbm.at[pl.ds(uid * 128, 128)], sw)

        def body(r, _):
            j0 = 2 * r
            j1 = 2 * r + 1
            phase(j0, g1a, g2a, sga, swa, r == 0)
            phase(j1, g1b, g2b, sgb, swb, r == 0)
            finish_phase(j0, g1a, g2a, sga, swa)
            finish_phase(j1, g1b, g2b, sgb, swb)
            return 0

        lax.fori_loop(0, UPT // 2, body, 0)

        @pl.when(cnt > 0)
        def _():
            pltpu.make_async_copy(g1a, o_hbm.at[pl.ds(0, 128)], swa).wait()

        @pl.when(cnt > 1)
        def _():
            pltpu.make_async_copy(g1b, o_hbm.at[pl.ds(0, 128)], swb).wait()

    return k(P1, P2, ia2d, ib2d)


def _sc_scatter(msg, ia2d, zeros_u, N, E):
    n_units = E // 128
    UPT = ((n_units + NW - 1) // NW + 7) // 8 * 8
    C = msg.shape[1]
    mesh = plsc.VectorSubcoreMesh(core_axis_name="c", subcore_axis_name="s")

    @functools.partial(
        pl.kernel,
        out_type=jax.ShapeDtypeStruct((2, N, 128), F32),
        mesh=mesh,
        scratch_types=[
            pltpu.VMEM((128,), jnp.int32),
            pltpu.VMEM((128,), jnp.int32),
            pltpu.VMEM((128, 64), F32),
            pltpu.VMEM((128, 64), F32),
            pltpu.VMEM((128, 128), F32),
            pltpu.VMEM_SHARED((N, 128), F32),
            pltpu.SemaphoreType.DMA,
            pltpu.SemaphoreType.DMA,
        ],
    )
    def k(msg_hbm, ia_hbm, z_hbm, out_hbm, iaa, iab, ma, mb, m_v, u_sh,
          sa, sb):
        cid = lax.axis_index("c")
        sid = lax.axis_index("s")
        wid = sid * 2 + cid
        start = wid * UPT
        cnt = jnp.minimum(jnp.maximum(n_units - start, 0), UPT)

        @pl.when(sid == 0)
        def _():
            pltpu.sync_copy(z_hbm, u_sh)

        zero = jnp.zeros((16,), F32)

        def zrow(r, _):
            for kk in range(8):
                m_v[r, pl.ds(16 * kk, 16)] = zero
            return 0

        lax.fori_loop(0, 128, zrow, 0)

        @pl.when(cnt > 0)
        def _():
            pltpu.async_copy(msg_hbm.at[pl.ds(start * 128, 128)], ma, sa)
            pltpu.async_copy(ia_hbm.at[start], iaa, sa)

        @pl.when(cnt > 1)
        def _():
            pltpu.async_copy(msg_hbm.at[pl.ds((start + 1) * 128, 128)], mb, sb)
            pltpu.async_copy(ia_hbm.at[start + 1], iab, sb)

        plsc.subcore_barrier()

        def consume(j, mv, iv, sem):
            uid = start + j

            @pl.when(j < cnt)
            def _():
                pltpu.make_async_copy(
                    msg_hbm.at[pl.ds(0, 128)], mv, sem).wait()
                pltpu.make_async_copy(ia_hbm.at[0], iv, sem).wait()

                def erow(r, _):
                    for kk in range(C // 16):
                        sl = pl.ds(16 * kk, 16)
                        m_v[r, sl] = mv[r, sl]
                    return 0

                lax.fori_loop(0, 128, erow, 0)
                pltpu.sync_copy(m_v, u_sh.at[iv], add=True)

                @pl.when(j + 2 < cnt)
                def _():
                    pltpu.async_copy(
                        msg_hbm.at[pl.ds((uid + 2) * 128, 128)], mv, sem)
                    pltpu.async_copy(ia_hbm.at[uid + 2], iv, sem)

        def body(r, _):
            consume(2 * r, ma, iaa, sa)
            consume(2 * r + 1, mb, iab, sb)
            return 0

        lax.fori_loop(0, UPT // 2, body, 0)
        plsc.subcore_barrier()

        @pl.when(sid == 0)
        def _():
            pltpu.sync_copy(u_sh, out_hbm.at[cid])

    return k(msg, ia2d, zeros_u)


def _embed(atom_fea, W_emb, b_emb, W1, W2):
    N, EMB = atom_fea.shape
    F = W_emb.shape[1]
    TF = W1.shape[1]
    BR = 2000

    def body(a_ref, we_ref, be_ref, w1_ref, w2_ref, x_ref, p1_ref, p2_ref):
        x = jnp.dot(a_ref[...], we_ref[...], preferred_element_type=F32)
        x = x + be_ref[0:1, :]
        x_ref[...] = x
        p1_ref[...] = jnp.dot(x, w1_ref[...], preferred_element_type=F32)
        p2_ref[...] = jnp.dot(x, w2_ref[...], preferred_element_type=F32)

    return pl.pallas_call(
        body,
        grid=(N // BR,),
        in_specs=[
            pl.BlockSpec((BR, EMB), lambda i: (i, 0)),
            pl.BlockSpec((EMB, F), lambda i: (0, 0)),
            pl.BlockSpec((8, F), lambda i: (0, 0)),
            pl.BlockSpec((F, TF), lambda i: (0, 0)),
            pl.BlockSpec((F, TF), lambda i: (0, 0)),
        ],
        out_specs=[
            pl.BlockSpec((BR, F), lambda i: (i, 0)),
            pl.BlockSpec((BR, TF), lambda i: (i, 0)),
            pl.BlockSpec((BR, TF), lambda i: (i, 0)),
        ],
        out_shape=[
            jax.ShapeDtypeStruct((N, F), F32),
            jax.ShapeDtypeStruct((N, TF), F32),
            jax.ShapeDtypeStruct((N, TF), F32),
        ],
    )(atom_fea, W_emb, b_emb, W1, W2)


def _stats(G12, nbr_fea, W3, b, E):
    TF = G12.shape[1]
    NB = nbr_fea.shape[1]
    BE = 2000

    def body(g_ref, nf_ref, w3_ref, b_ref, s1_ref, s2_ref):
        t = (g_ref[...]
             + jnp.dot(nf_ref[...], w3_ref[...], preferred_element_type=F32)
             + b_ref[0:1, :])
        p1 = jnp.sum(t.reshape(BE // 8, 8, TF), axis=0)
        p2 = jnp.sum((t * t).reshape(BE // 8, 8, TF), axis=0)

        @pl.when(pl.program_id(0) == 0)
        def _():
            s1_ref[...] = jnp.zeros_like(s1_ref)
            s2_ref[...] = jnp.zeros_like(s2_ref)

        s1_ref[...] += p1
        s2_ref[...] += p2

    return pl.pallas_call(
        body,
        grid=(E // BE,),
        in_specs=[
            pl.BlockSpec((BE, TF), lambda i: (i, 0)),
            pl.BlockSpec((BE, NB), lambda i: (i, 0)),
            pl.BlockSpec((NB, TF), lambda i: (0, 0)),
            pl.BlockSpec((8, TF), lambda i: (0, 0)),
        ],
        out_specs=[
            pl.BlockSpec((8, TF), lambda i: (0, 0)),
            pl.BlockSpec((8, TF), lambda i: (0, 0)),
        ],
        out_shape=[
            jax.ShapeDtypeStruct((8, TF), F32),
            jax.ShapeDtypeStruct((8, TF), F32),
        ],
    )(G12, nbr_fea, W3, b)


def _gate(G12, nbr_fea, W3, b, scale, shift, E):
    TF = G12.shape[1]
    NB = nbr_fea.shape[1]
    F = TF // 2
    BE = 2000

    def body(g_ref, nf_ref, w3_ref, b_ref, sc_ref, sh_ref, m_ref):
        t = (g_ref[...]
             + jnp.dot(nf_ref[...], w3_ref[...], preferred_element_type=F32)
             + b_ref[0:1, :])
        t = t * sc_ref[0:1, :] + sh_ref[0:1, :]
        LOG2E = 1.4426950408889634
        LN2 = 0.6931471805599453
        f = t[:, :F]
        c = t[:, F:]
        filt = 1.0 / (1.0 + jnp.exp2(-LOG2E * f))
        core = jnp.maximum(c, 0.0) + LN2 * jnp.log2(
            1.0 + jnp.exp2(-LOG2E * jnp.abs(c)))
        m_ref[...] = filt * core

    return pl.pallas_call(
        body,
        grid=(E // BE,),
        in_specs=[
            pl.BlockSpec((BE, TF), lambda i: (i, 0)),
            pl.BlockSpec((BE, NB), lambda i: (i, 0)),
            pl.BlockSpec((NB, TF), lambda i: (0, 0)),
            pl.BlockSpec((8, TF), lambda i: (0, 0)),
            pl.BlockSpec((8, TF), lambda i: (0, 0)),
            pl.BlockSpec((8, TF), lambda i: (0, 0)),
        ],
        out_specs=pl.BlockSpec((BE, F), lambda i: (i, 0)),
        out_shape=jax.ShapeDtypeStruct((E, F), F32),
    )(G12, nbr_fea, W3, b, scale, shift)


def _finish(Uparts, x, g2, b2, W1, W2, last):
    N, F = x.shape
    TF = W1.shape[1]

    def body(u_ref, x_ref, g_ref, b_ref, w1_ref, w2_ref, *outs):
        u = u_ref[0, :, :F] + u_ref[1, :, :F]
        mean = jnp.mean(u, axis=0, keepdims=True)
        var = jnp.mean(u * u, axis=0, keepdims=True) - mean * mean
        un = g_ref[0:1, :] * (u - mean) * jax.lax.rsqrt(var + EPS) + b_ref[0:1, :]
        xn = jax.nn.softplus(x_ref[...] + un)
        outs[0][...] = xn
        if not last:
            outs[1][...] = jnp.dot(xn, w1_ref[...], preferred_element_type=F32)
            outs[2][...] = jnp.dot(xn, w2_ref[...], preferred_element_type=F32)

    out_shape = [jax.ShapeDtypeStruct((N, F), F32)]
    if not last:
        out_shape += [jax.ShapeDtypeStruct((N, TF), F32),
                      jax.ShapeDtypeStruct((N, TF), F32)]
    return pl.pallas_call(body, out_shape=out_shape)(Uparts, x, g2, b2, W1, W2)


def kernel(atom_fea, nbr_fea, self_idx, nbr_idx, W_emb, b_emb, fc_W, fc_b,
           bn1_g, bn1_b, bn2_g, bn2_b):
    N = atom_fea.shape[0]
    E = nbr_fea.shape[0]
    F = W_emb.shape[1]
    TF = 2 * F
    n_layers = fc_W.shape[0]

    W1 = fc_W[:, :F, :]
    W2 = fc_W[:, F:TF, :]
    W3 = fc_W[:, TF:, :]

    def row8(v):
        return jnp.broadcast_to(v.reshape(1, -1), (8, v.shape[0])).astype(F32)

    ia2d = self_idx.reshape(E // 128, 128)
    ib2d = nbr_idx.reshape(E // 128, 128)
    zeros_u = jnp.zeros((N, 128), F32)

    x, P1, P2 = _embed(atom_fea, W_emb, row8(b_emb), W1[0], W2[0])

    for i in range(n_layers):
        G12 = _sc_gather(P1, P2, ia2d, ib2d, E)
        s1, s2 = _stats(G12, nbr_fea, W3[i], row8(fc_b[i]), E)
        mean = jnp.sum(s1, axis=0) / E
        var = jnp.sum(s2, axis=0) / E - mean * mean
        scale = bn1_g[i] * jax.lax.rsqrt(var + EPS)
        shift = bn1_b[i] - mean * scale
        msg = _gate(G12, nbr_fea, W3[i], row8(fc_b[i]),
                    row8(scale), row8(shift), E)
        Uparts = _sc_scatter(msg, ia2d, zeros_u, N, E)
        last = i == n_layers - 1
        nxt = 0 if last else i + 1
        outs = _finish(Uparts, x, row8(bn2_g[i]), row8(bn2_b[i]),
                       W1[nxt], W2[nxt], last)
        if last:
            x = outs[0]
        else:
            x, P1, P2 = outs
    return x


# speedup vs baseline: 2.9495x; 1.1143x over previous
import functools

import jax
import jax.numpy as jnp
from jax import lax
from jax.experimental import pallas as pl
from jax.experimental.pallas import tpu as pltpu
from jax.experimental.pallas import tpu_sc as plsc

F32 = jnp.float32
NW = 32
EPS = 1e-5


def _sc_gather(P1, P2, ia2d, ib2d, E):
    n_units = E // 128
    UPT = ((n_units + NW - 1) // NW + 7) // 8 * 8
    C = P1.shape[1]
    mesh = plsc.VectorSubcoreMesh(core_axis_name="c", subcore_axis_name="s")

    @functools.partial(
        pl.kernel,
        out_type=jax.ShapeDtypeStruct((E, C), F32),
        mesh=mesh,
        scratch_types=[
            pltpu.VMEM((UPT, 128), jnp.int32),
            pltpu.VMEM((UPT, 128), jnp.int32),
            pltpu.VMEM((128, C), F32),
            pltpu.VMEM((128, C), F32),
            pltpu.VMEM((128, C), F32),
            pltpu.VMEM((128, C), F32),
            pltpu.VMEM((128, C), F32),
            pltpu.VMEM((128, C), F32),
            pltpu.SemaphoreType.DMA,
            pltpu.SemaphoreType.DMA,
            pltpu.SemaphoreType.DMA,
            pltpu.SemaphoreType.DMA,
            pltpu.SemaphoreType.DMA,
            pltpu.SemaphoreType.DMA,
        ],
    )
    def k(p1_hbm, p2_hbm, ia_hbm, ib_hbm, o_hbm,
          ia_v, ib_v, g1a, g2a, g1b, g2b, g1c, g2c,
          sga, sgb, sgc, swa, swb, swc):
        wid = lax.axis_index("s") * 2 + lax.axis_index("c")
        start = wid * UPT
        cnt = jnp.minimum(jnp.maximum(n_units - start, 0), UPT)

        @pl.when(cnt > 0)
        def _():
            pltpu.sync_copy(ia_hbm.at[pl.ds(start, UPT)], ia_v)
            pltpu.sync_copy(ib_hbm.at[pl.ds(start, UPT)], ib_v)

        def addto(dst, src):
            def arow(r, _):
                for kk in range(C // 16):
                    sl = pl.ds(16 * kk, 16)
                    dst[r, sl] = dst[r, sl] + src[r, sl]
                return 0
            lax.fori_loop(0, 128, arow, 0)

        def phase(j, g1, g2, sg, sw, first):
            uid = start + j

            @pl.when(j < cnt)
            def _():
                @pl.when(jnp.logical_not(first))
                def _():
                    pltpu.make_async_copy(
                        g1, o_hbm.at[pl.ds(0, 128)], sw).wait()
                pltpu.async_copy(p1_hbm.at[ia_v.at[j]], g1, sg)
                pltpu.async_copy(p2_hbm.at[ib_v.at[j]], g2, sg)

        def finish_phase(j, g1, g2, sg, sw):
            uid = start + j

            @pl.when(j < cnt)
            def _():
                pltpu.make_async_copy(p1_hbm.at[ia_v.at[j]], g1, sg).wait()
                pltpu.make_async_copy(p2_hbm.at[ib_v.at[j]], g2, sg).wait()
                addto(g1, g2)
                pltpu.async_copy(g1, o_hbm.at[pl.ds(uid * 128, 128)], sw)

        bufs = [(g1a, g2a, sga, swa), (g1b, g2b, sgb, swb),
                (g1c, g2c, sgc, swc)]

        def body(r, _):
            for kk in range(3):
                phase(3 * r + kk, *bufs[kk], r == 0)
            for kk in range(3):
                finish_phase(3 * r + kk, *bufs[kk])
            return 0

        lax.fori_loop(0, (UPT + 2) // 3, body, 0)

        for kk in range(3):
            @pl.when(cnt > kk)
            def _(kk=kk):
                pltpu.make_async_copy(
                    bufs[kk][0], o_hbm.at[pl.ds(0, 128)], bufs[kk][3]).wait()

    return k(P1, P2, ia2d, ib2d)


def _sc_scatter(msg, ia2d, zeros_u, N, E):
    n_units = E // 128
    UPT = ((n_units + NW - 1) // NW + 7) // 8 * 8
    C = msg.shape[1]
    mesh = plsc.VectorSubcoreMesh(core_axis_name="c", subcore_axis_name="s")

    @functools.partial(
        pl.kernel,
        out_type=jax.ShapeDtypeStruct((2, N, 128), F32),
        mesh=mesh,
        scratch_types=[
            pltpu.VMEM((128,), jnp.int32),
            pltpu.VMEM((128,), jnp.int32),
            pltpu.VMEM((128, 64), F32),
            pltpu.VMEM((128, 64), F32),
            pltpu.VMEM((128, 128), F32),
            pltpu.VMEM_SHARED((N, 128), F32),
            pltpu.SemaphoreType.DMA,
            pltpu.SemaphoreType.DMA,
        ],
    )
    def k(msg_hbm, ia_hbm, z_hbm, out_hbm, iaa, iab, ma, mb, m_v, u_sh,
          sa, sb):
        cid = lax.axis_index("c")
        sid = lax.axis_index("s")
        wid = sid * 2 + cid
        start = wid * UPT
        cnt = jnp.minimum(jnp.maximum(n_units - start, 0), UPT)

        @pl.when(sid == 0)
        def _():
            pltpu.sync_copy(z_hbm, u_sh)

        zero = jnp.zeros((16,), F32)

        def zrow(r, _):
            for kk in range(8):
                m_v[r, pl.ds(16 * kk, 16)] = zero
            return 0

        lax.fori_loop(0, 128, zrow, 0)

        @pl.when(cnt > 0)
        def _():
            pltpu.async_copy(msg_hbm.at[pl.ds(start * 128, 128)], ma, sa)
            pltpu.async_copy(ia_hbm.at[start], iaa, sa)

        @pl.when(cnt > 1)
        def _():
            pltpu.async_copy(msg_hbm.at[pl.ds((start + 1) * 128, 128)], mb, sb)
            pltpu.async_copy(ia_hbm.at[start + 1], iab, sb)

        plsc.subcore_barrier()

        def consume(j, mv, iv, sem):
            uid = start + j

            @pl.when(j < cnt)
            def _():
                pltpu.make_async_copy(
                    msg_hbm.at[pl.ds(0, 128)], mv, sem).wait()
                pltpu.make_async_copy(ia_hbm.at[0], iv, sem).wait()

                def erow(r, _):
                    for kk in range(C // 16):
                        sl = pl.ds(16 * kk, 16)
                        m_v[r, sl] = mv[r, sl]
                    return 0

                lax.fori_loop(0, 128, erow, 0)
                pltpu.sync_copy(m_v, u_sh.at[iv], add=True)

                @pl.when(j + 2 < cnt)
                def _():
                    pltpu.async_copy(
                        msg_hbm.at[pl.ds((uid + 2) * 128, 128)], mv, sem)
                    pltpu.async_copy(ia_hbm.at[uid + 2], iv, sem)

        def body(r, _):
            consume(2 * r, ma, iaa, sa)
            consume(2 * r + 1, mb, iab, sb)
            return 0

        lax.fori_loop(0, UPT // 2, body, 0)
        plsc.subcore_barrier()

        @pl.when(sid == 0)
        def _():
            pltpu.sync_copy(u_sh, out_hbm.at[cid])

    return k(msg, ia2d, zeros_u)


def _embed(atom_fea, W_emb, b_emb, W1, W2):
    N, EMB = atom_fea.shape
    F = W_emb.shape[1]
    TF = W1.shape[1]
    BR = 2000

    def body(a_ref, we_ref, be_ref, w1_ref, w2_ref, x_ref, p1_ref, p2_ref):
        x = jnp.dot(a_ref[...], we_ref[...], preferred_element_type=F32)
        x = x + be_ref[0:1, :]
        x_ref[...] = x
        p1_ref[...] = jnp.dot(x, w1_ref[...], preferred_element_type=F32)
        p2_ref[...] = jnp.dot(x, w2_ref[...], preferred_element_type=F32)

    return pl.pallas_call(
        body,
        grid=(N // BR,),
        in_specs=[
            pl.BlockSpec((BR, EMB), lambda i: (i, 0)),
            pl.BlockSpec((EMB, F), lambda i: (0, 0)),
            pl.BlockSpec((8, F), lambda i: (0, 0)),
            pl.BlockSpec((F, TF), lambda i: (0, 0)),
            pl.BlockSpec((F, TF), lambda i: (0, 0)),
        ],
        out_specs=[
            pl.BlockSpec((BR, F), lambda i: (i, 0)),
            pl.BlockSpec((BR, TF), lambda i: (i, 0)),
            pl.BlockSpec((BR, TF), lambda i: (i, 0)),
        ],
        out_shape=[
            jax.ShapeDtypeStruct((N, F), F32),
            jax.ShapeDtypeStruct((N, TF), F32),
            jax.ShapeDtypeStruct((N, TF), F32),
        ],
    )(atom_fea, W_emb, b_emb, W1, W2)


def _stats(G12, nbr_fea, W3, b, E):
    TF = G12.shape[1]
    NB = nbr_fea.shape[1]
    BE = 8000

    def body(g_ref, nf_ref, w3_ref, b_ref, s1_ref, s2_ref):
        t = (g_ref[...]
             + jnp.dot(nf_ref[...], w3_ref[...], preferred_element_type=F32)
             + b_ref[0:1, :])
        p1 = jnp.sum(t.reshape(BE // 8, 8, TF), axis=0)
        p2 = jnp.sum((t * t).reshape(BE // 8, 8, TF), axis=0)

        @pl.when(pl.program_id(0) == 0)
        def _():
            s1_ref[...] = jnp.zeros_like(s1_ref)
            s2_ref[...] = jnp.zeros_like(s2_ref)

        s1_ref[...] += p1
        s2_ref[...] += p2

    return pl.pallas_call(
        body,
        grid=(E // BE,),
        in_specs=[
            pl.BlockSpec((BE, TF), lambda i: (i, 0)),
            pl.BlockSpec((BE, NB), lambda i: (i, 0)),
            pl.BlockSpec((NB, TF), lambda i: (0, 0)),
            pl.BlockSpec((8, TF), lambda i: (0, 0)),
        ],
        out_specs=[
            pl.BlockSpec((8, TF), lambda i: (0, 0)),
            pl.BlockSpec((8, TF), lambda i: (0, 0)),
        ],
        out_shape=[
            jax.ShapeDtypeStruct((8, TF), F32),
            jax.ShapeDtypeStruct((8, TF), F32),
        ],
    )(G12, nbr_fea, W3, b)


def _gate(G12, nbr_fea, W3, b, scale, shift, E):
    TF = G12.shape[1]
    NB = nbr_fea.shape[1]
    F = TF // 2
    BE = 2000

    def body(g_ref, nf_ref, w3_ref, b_ref, sc_ref, sh_ref, m_ref):
        t = (g_ref[...]
             + jnp.dot(nf_ref[...], w3_ref[...], preferred_element_type=F32)
             + b_ref[0:1, :])
        t = t * sc_ref[0:1, :] + sh_ref[0:1, :]
        LOG2E = 1.4426950408889634
        LN2 = 0.6931471805599453
        f = t[:, :F]
        c = t[:, F:]
        filt = 1.0 / (1.0 + jnp.exp2(-LOG2E * f))
        core = jnp.maximum(c, 0.0) + LN2 * jnp.log2(
            1.0 + jnp.exp2(-LOG2E * jnp.abs(c)))
        m_ref[...] = filt * core

    return pl.pallas_call(
        body,
        grid=(E // BE,),
        in_specs=[
            pl.BlockSpec((BE, TF), lambda i: (i, 0)),
            pl.BlockSpec((BE, NB), lambda i: (i, 0)),
            pl.BlockSpec((NB, TF), lambda i: (0, 0)),
            pl.BlockSpec((8, TF), lambda i: (0, 0)),
            pl.BlockSpec((8, TF), lambda i: (0, 0)),
            pl.BlockSpec((8, TF), lambda i: (0, 0)),
        ],
        out_specs=pl.BlockSpec((BE, F), lambda i: (i, 0)),
        out_shape=jax.ShapeDtypeStruct((E, F), F32),
    )(G12, nbr_fea, W3, b, scale, shift)


def _finish(Uparts, x, g2, b2, W1, W2, last):
    N, F = x.shape
    TF = W1.shape[1]

    def body(u_ref, x_ref, g_ref, b_ref, w1_ref, w2_ref, *outs):
        u = u_ref[0, :, :F] + u_ref[1, :, :F]
        mean = jnp.mean(u, axis=0, keepdims=True)
        var = jnp.mean(u * u, axis=0, keepdims=True) - mean * mean
        un = g_ref[0:1, :] * (u - mean) * jax.lax.rsqrt(var + EPS) + b_ref[0:1, :]
        xn = jax.nn.softplus(x_ref[...] + un)
        outs[0][...] = xn
        if not last:
            outs[1][...] = jnp.dot(xn, w1_ref[...], preferred_element_type=F32)
            outs[2][...] = jnp.dot(xn, w2_ref[...], preferred_element_type=F32)

    out_shape = [jax.ShapeDtypeStruct((N, F), F32)]
    if not last:
        out_shape += [jax.ShapeDtypeStruct((N, TF), F32),
                      jax.ShapeDtypeStruct((N, TF), F32)]
    return pl.pallas_call(body, out_shape=out_shape)(Uparts, x, g2, b2, W1, W2)


def kernel(atom_fea, nbr_fea, self_idx, nbr_idx, W_emb, b_emb, fc_W, fc_b,
           bn1_g, bn1_b, bn2_g, bn2_b):
    N = atom_fea.shape[0]
    E = nbr_fea.shape[0]
    F = W_emb.shape[1]
    TF = 2 * F
    n_layers = fc_W.shape[0]

    W1 = fc_W[:, :F, :]
    W2 = fc_W[:, F:TF, :]
    W3 = fc_W[:, TF:, :]

    def row8(v):
        return jnp.broadcast_to(v.reshape(1, -1), (8, v.shape[0])).astype(F32)

    ia2d = self_idx.reshape(E // 128, 128)
    ib2d = nbr_idx.reshape(E // 128, 128)
    zeros_u = jnp.zeros((N, 128), F32)

    x, P1, P2 = _embed(atom_fea, W_emb, row8(b_emb), W1[0], W2[0])

    for i in range(n_layers):
        G12 = _sc_gather(P1, P2, ia2d, ib2d, E)
        s1, s2 = _stats(G12, nbr_fea, W3[i], row8(fc_b[i]), E)
        mean = jnp.sum(s1, axis=0) / E
        var = jnp.sum(s2, axis=0) / E - mean * mean
        scale = bn1_g[i] * jax.lax.rsqrt(var + EPS)
        shift = bn1_b[i] - mean * scale
        msg = _gate(G12, nbr_fea, W3[i], row8(fc_b[i]),
                    row8(scale), row8(shift), E)
        Uparts = _sc_scatter(msg, ia2d, zeros_u, N, E)
        last = i == n_layers - 1
        nxt = 0 if last else i + 1
        outs = _finish(Uparts, x, row8(bn2_g[i]), row8(bn2_b[i]),
                       W1[nxt], W2[nxt], last)
        if last:
            x = outs[0]
        else:
            x, P1, P2 = outs
    return x
